# Initial kernel scaffold; baseline (speedup 1.0000x reference)
import jax
import jax.numpy as jnp
from jax.experimental import pallas as pl


def kernel(Q, K, V, attn_mask, wq0, bq0, wq1, bq1, wq2, bq2, wk0, bk0, wk1, bk1, wk2, bk2, gamma, beta, proj_q_w, proj_k_w, proj_back_q_w, proj_back_k_w):
    raise NotImplementedError("write your pallas kernel here")


# speedup vs baseline: 5.6873x; 5.6873x over previous
import functools
import math

import jax
import jax.numpy as jnp
import numpy as np
from jax.experimental import pallas as pl
from jax.experimental.pallas import tpu as pltpu

_B, _H, _L, _DK = 1, 12, 2048, 64
_C = _H * _DK


_NTAP = 26
_FIRST_TAPS = (0, 1, 2, 5, 8, 17)
_STAGE_ENDS = ((0, None), (1, 0), (4, None), (7, 1), (16, None), (25, 2))


def _conv_body(x_ref, gamma_ref, beta_ref, wproj_ref, w_ref,
               p_ref, km_ref, curp, acc):
  j = pl.program_id(1)

  @pl.when(j == 0)
  def _init():
    curp[...] = x_ref[0]

  r = jnp.where(j < 2, 0,
      jnp.where(j < 5, 3 - j,
      jnp.where(j < 8, 6 - j,
      jnp.where(j < 17, 12 - j, 21 - j))))
  rolled = pltpu.roll(curp[...], jnp.where(r < 0, r + _L, r), axis=1)
  col = jax.lax.broadcasted_iota(jnp.int32, (_C, _L), 1)
  xs = jnp.where((col >= r) & (col < _L + r), rolled, 0.0)
  t = jnp.dot(w_ref[0, 0], xs, preferred_element_type=jnp.float32)
  first = ((j == 0) | (j == 1) | (j == 2) | (j == 5) | (j == 8) | (j == 17))

  @pl.when(first)
  def _set():
    acc[...] = t

  @pl.when(jnp.logical_not(first))
  def _add():
    acc[...] += t

  def finalize(side_idx):
    a = acc[...]
    mu = jnp.mean(a, axis=1, keepdims=True)
    m2 = jnp.mean(a * a, axis=1, keepdims=True)
    var = m2 - mu * mu
    scale = jax.lax.rsqrt(var + 1e-5) * gamma_ref[...]
    shift = beta_ref[...] - mu * scale
    yh = a * scale + shift
    z = jnp.where(yh > 0, yh, jnp.exp(yh) - 1.0)
    if side_idx is None:
      curp[...] = z
    else:
      p = jnp.dot(z, wproj_ref[0], preferred_element_type=jnp.float32)
      p_ref[0, side_idx] = p
      for hh in range(4):
        base = 192 * hh
        pm = (p[base:base + 64] + p[base + 64:base + 128] +
              p[base + 128:base + 192]) * (1.0 / 3.0)
        km_ref[0, side_idx, 64 * hh:64 * hh + 64] = pm

  for jend, side_idx in _STAGE_ENDS:
    @pl.when(j == jend)
    def _fin(side_idx=side_idx):
      finalize(side_idx)


def _attn_body(qt_ref, km_ref, v_ref, pbq_ref, pbk_ref, attn_ref, ctx_ref):
  s = jnp.sum(pbq_ref[...] * pbk_ref[...], keepdims=True) * 0.125
  a = qt_ref[0] * s
  m = km_ref[0]
  logits = a * m
  mx = jnp.max(logits, axis=1, keepdims=True)
  e = jnp.exp(logits - mx)
  ssum = jnp.sum(e, axis=1, keepdims=True)
  attn = e / ssum
  attn_ref[0] = attn
  ctx_ref[0] = jnp.dot(attn, v_ref[0], preferred_element_type=jnp.float32)


def kernel(Q, K, V, attn_mask, wq0, bq0, wq1, bq1, wq2, bq2,
           wk0, bk0, wk1, bk1, wk2, bk2, gamma, beta,
           proj_q_w, proj_k_w, proj_back_q_w, proj_back_k_w):
  b, h, l, d_k = Q.shape
  del attn_mask

  x0 = jnp.stack([Q.reshape(_C, _L), K.reshape(_C, _L)])

  def prep_w(wq, wk):
    return jnp.stack([jnp.moveaxis(wq, 2, 0), jnp.moveaxis(wk, 2, 0)])

  w1 = prep_w(wq0, wk0)
  w3 = prep_w(wq1, wk1)
  w9 = prep_w(wq2, wk2)
  wcat = jnp.concatenate([w1, w1, w3, w3, w9, w9], axis=1)
  del bq0, bq1, bq2, bk0, bk1, bk2

  eye = jnp.eye(32, dtype=jnp.float32)
  def band(w):
    m = eye[:, None, :] * w[0][None, :, None]
    return m.reshape(2048, 32)
  wproj = jnp.stack([band(proj_q_w), band(proj_k_w)])

  p, km = pl.pallas_call(
      _conv_body,
      grid=(2, _NTAP),
      in_specs=[
          pl.BlockSpec((1, _C, _L), lambda i, j: (i, 0, 0)),
          pl.BlockSpec((_C, 1), lambda i, j: (0, 0)),
          pl.BlockSpec((_C, 1), lambda i, j: (0, 0)),
          pl.BlockSpec((1, 2048, 32), lambda i, j: (i, 0, 0)),
          pl.BlockSpec((1, 1, _C, _C), lambda i, j: (i, j, 0, 0)),
      ],
      out_specs=[
          pl.BlockSpec((1, 3, _C, 32), lambda i, j: (i, 0, 0, 0)),
          pl.BlockSpec((1, 3, 256, 32), lambda i, j: (i, 0, 0, 0)),
      ],
      out_shape=[
          jax.ShapeDtypeStruct((2, 3, _C, 32), jnp.float32),
          jax.ShapeDtypeStruct((2, 3, 256, 32), jnp.float32),
      ],
      scratch_shapes=[
          pltpu.VMEM((_C, _L), jnp.float32),
          pltpu.VMEM((_C, _L), jnp.float32),
      ],
      compiler_params=pltpu.CompilerParams(
          dimension_semantics=("arbitrary", "arbitrary")),
  )(x0, gamma[:, None], beta[:, None], wproj, wcat)

  qp = p[0].reshape(_H, 3 * _L)
  k_mean = km[1].reshape(_H, _L)

  q_top = jax.lax.top_k(qp, _L)[0]

  qb = 256
  attn, ctx = pl.pallas_call(
      _attn_body,
      grid=(_H, _L // qb),
      in_specs=[
          pl.BlockSpec((1, qb, 1), lambda hh, j: (hh, j, 0)),
          pl.BlockSpec((1, 1, _L), lambda hh, j: (hh, 0, 0)),
          pl.BlockSpec((1, _L, _DK), lambda hh, j: (hh, 0, 0)),
          pl.BlockSpec((1, _DK), lambda hh, j: (0, 0)),
          pl.BlockSpec((1, _DK), lambda hh, j: (0, 0)),
      ],
      out_specs=[
          pl.BlockSpec((1, qb, _L), lambda hh, j: (hh, j, 0)),
          pl.BlockSpec((1, qb, _DK), lambda hh, j: (hh, j, 0)),
      ],
      out_shape=[
          jax.ShapeDtypeStruct((_H, _L, _L), jnp.float32),
          jax.ShapeDtypeStruct((_H, _L, _DK), jnp.float32),
      ],
      compiler_params=pltpu.CompilerParams(
          dimension_semantics=("arbitrary", "arbitrary")),
  )(q_top[:, :, None], k_mean[:, None, :], V[0],
    proj_back_q_w.reshape(1, _DK), proj_back_k_w.reshape(1, _DK))

  return (ctx.reshape(b, h, l, d_k), attn.reshape(b, h, l, l))
